# baseline (speedup 1.0000x reference)
import functools

import jax
import jax.numpy as jnp
from jax import lax
from jax.experimental import pallas as pl
from jax.experimental.pallas import tpu as pltpu
from jax.experimental.pallas import tpu_sc as plsc

_TIMESTEP = 5
_VTH = 1.0
_TAU = 0.5
_BETA = 0.2

_B = 80
_C, _H, _W = 192, 32, 32
_P = _C * _H * _W
_K = int(_BETA * _P)
_BS = _B // _TIMESTEP
_HW = _H * _W

_NW = 32
_CHUNK = 64
_NCH = _HW // _CHUNK
_VPC = _CHUNK * _C // 16

_HW_CHUNK = 512
_NCHUNK = _HW // _HW_CHUNK


def _iota16():
    return lax.broadcasted_iota(jnp.int32, (16,), 0)


def _suffix_incl(vec):
    rev = lax.rev(vec, (0,))
    return lax.rev(plsc.cumsum(rev), (0,))


def _find_bin(vec, base_count, k):
    suf = _suffix_incl(vec)
    cond = (base_count + suf) >= k
    lane = jnp.max(jnp.where(cond, _iota16(), jnp.int32(-1)))
    above = jnp.sum(jnp.where(_iota16() == lane, suf - vec, 0.0))
    return lane, base_count + above


def _scan_hists(hist, hist_c, k):
    def body(t, carry):
        running, found_v, run_at = carry
        v = 255 - t
        s = jnp.sum(hist_c[pl.ds(v * 16, 16)])
        new = running + s
        hit = jnp.logical_and(running < k, new >= k)
        found_v = jnp.where(hit, v, found_v)
        run_at = jnp.where(hit, running, run_at)
        return new, found_v, run_at

    _, vc, run_c = lax.fori_loop(
        0, 256, body, (jnp.float32(0.0), jnp.int32(0), jnp.float32(0.0)))
    lane_c, above_c = _find_bin(hist_c[pl.ds(vc * 16, 16)], run_c, k)
    bc = vc * 16 + lane_c
    lane_f, above_f = _find_bin(hist[pl.ds(bc * 16, 16)], above_c, k)
    return bc * 16 + lane_f, above_f


def _sc_thresholds(xi):
    mesh = plsc.VectorSubcoreMesh(core_axis_name="c", subcore_axis_name="s")

    @functools.partial(
        pl.kernel, mesh=mesh,
        compiler_params=pltpu.CompilerParams(needs_layout_passes=False),
        out_type=jax.ShapeDtypeStruct((_B, 16), jnp.int32),
        scratch_types=[
            pltpu.VMEM((2, _CHUNK, _C), jnp.int32),
            pltpu.VMEM((65536,), jnp.float32),
            pltpu.VMEM((4096,), jnp.float32),
            pltpu.VMEM((16,), jnp.int32),
            pltpu.SemaphoreType.DMA,
            pltpu.SemaphoreType.DMA,
        ],
    )
    def sck(xi_hbm, o_hbm, buf, hist, hist_c, ov, sem0, sem1):
        wid = lax.axis_index("s") * 2 + lax.axis_index("c")
        sems = (sem0, sem1)
        zf = jnp.zeros((16,), jnp.float32)

        def zero_hists():
            def zb(i, _):
                hist[pl.ds(i * 16, 16)] = zf
                return ()
            lax.fori_loop(0, 4096, zb, ())

            def zc(i, _):
                hist_c[pl.ds(i * 16, 16)] = zf
                return ()
            lax.fori_loop(0, 256, zc, ())

        def row_pass(r, fine_sel, k):
            zero_hists()
            ones = jnp.ones((16,), jnp.float32)

            def start(c):
                return pltpu.async_copy(
                    xi_hbm.at[r, pl.ds(c * _CHUNK, _CHUNK), :],
                    buf.at[c % 2], sems[c % 2])

            h = {0: start(0)}
            for c in range(_NCH):
                h[c].wait()
                if c + 1 < _NCH:
                    h[c + 1] = start(c + 1)
                cur = c % 2

                def ibody(i, _):
                    for j in range(_C // 16):
                        v = buf[cur, i, pl.ds(j * 16, 16)]
                        key = jnp.where(v < 0, v ^ jnp.int32(0x7FFFFFFF), v)
                        b1 = (key >> 16) + jnp.int32(32768)
                        is_a = fine_sel < jnp.int32(0)
                        bins = jnp.where(is_a, b1, key & jnp.int32(0xFFFF))
                        m = jnp.logical_or(is_a, b1 == fine_sel)
                        plsc.addupdate_scatter(hist, (bins,), ones, mask=m)
                        plsc.addupdate_scatter(hist_c, (bins >> 4,), ones,
                                               mask=m)
                    return ()
                lax.fori_loop(0, _CHUNK, ibody, ())
            return _scan_hists(hist, hist_c, k)

        def process_row(i, _):
            r = wid + _NW * i

            @pl.when(r < _B)
            def _():
                b1, above_a = row_pass(r, jnp.int32(-1), jnp.float32(_K))
                lo, _above2 = row_pass(r, b1, jnp.float32(_K) - above_a)
                kth = ((b1 - 32768) << 16) | lo
                ov[...] = jnp.broadcast_to(kth, (16,))
                pltpu.sync_copy(ov, o_hbm.at[r])
            return ()

        lax.fori_loop(0, 3, process_row, ())

    return sck(xi)


def _lif_body(thr_ref, x_ref, o_ref):
    j = pl.program_id(0)
    u = jnp.zeros((_HW_CHUNK, _C), jnp.float32)
    for t in range(_TIMESTEP):
        xt = x_ref[t, 0]
        kth_f = thr_ref[t * _BS + j]
        mask = (xt >= kth_f).astype(jnp.float32)
        spk_prev = (u > _VTH).astype(jnp.float32)
        u = _TAU * u * (1.0 - spk_prev) + xt
        s = (u > _VTH).astype(jnp.float32)
        o_ref[t, 0] = s * mask


def kernel(x):
    xp = x.transpose(0, 2, 3, 1).reshape(_B, _HW, _C)
    xi = jax.lax.bitcast_convert_type(xp, jnp.int32)
    kth = _sc_thresholds(xi)[:, 0]
    kth_f = jax.lax.bitcast_convert_type(
        jnp.where(kth < 0, kth ^ jnp.int32(0x7FFFFFFF), kth), jnp.float32)

    x4 = xp.reshape(_TIMESTEP, _BS, _HW, _C)
    out = pl.pallas_call(
        _lif_body,
        grid_spec=pltpu.PrefetchScalarGridSpec(
            num_scalar_prefetch=1,
            grid=(_BS, _NCHUNK),
            in_specs=[pl.BlockSpec((_TIMESTEP, 1, _HW_CHUNK, _C),
                                   lambda j, c, *_: (0, j, c, 0))],
            out_specs=pl.BlockSpec((_TIMESTEP, 1, _HW_CHUNK, _C),
                                   lambda j, c, *_: (0, j, c, 0)),
        ),
        out_shape=jax.ShapeDtypeStruct((_TIMESTEP, _BS, _HW, _C),
                                       jnp.float32),
    )(kth_f, x4)
    return out.reshape(_B, _H, _W, _C).transpose(0, 3, 1, 2)


# speedup vs baseline: 1.1368x; 1.1368x over previous
import functools

import jax
import jax.numpy as jnp
from jax import lax
from jax.experimental import pallas as pl
from jax.experimental.pallas import tpu as pltpu
from jax.experimental.pallas import tpu_sc as plsc

_TIMESTEP = 5
_VTH = 1.0
_TAU = 0.5
_BETA = 0.2

_B = 80
_C, _H, _W = 192, 32, 32
_P = _C * _H * _W
_K = int(_BETA * _P)
_BS = _B // _TIMESTEP
_HW = _H * _W

_NW = 32
_CHUNK = 64
_NCH = _HW // _CHUNK
_VPC = _CHUNK * _C // 16

_HW_CHUNK = 512
_NCHUNK = _HW // _HW_CHUNK


def _iota16():
    return lax.broadcasted_iota(jnp.int32, (16,), 0)


def _suffix_incl(vec):
    rev = lax.rev(vec, (0,))
    return lax.rev(plsc.cumsum(rev), (0,))


def _find_bin(vec, base_count, k):
    suf = _suffix_incl(vec)
    cond = (base_count + suf) >= k
    lane = jnp.max(jnp.where(cond, _iota16(), jnp.int32(-1)))
    above = jnp.sum(jnp.where(_iota16() == lane, suf - vec, 0.0))
    return lane, base_count + above


def _scan_hists(hist, hist_c, k):
    def body(t, carry):
        running, found_v, run_at = carry
        v = 255 - t
        s = jnp.sum(hist_c[pl.ds(v * 16, 16)])
        new = running + s
        hit = jnp.logical_and(running < k, new >= k)
        found_v = jnp.where(hit, v, found_v)
        run_at = jnp.where(hit, running, run_at)
        return new, found_v, run_at

    _, vc, run_c = lax.fori_loop(
        0, 256, body, (jnp.float32(0.0), jnp.int32(0), jnp.float32(0.0)))
    lane_c, above_c = _find_bin(hist_c[pl.ds(vc * 16, 16)], run_c, k)
    bc = vc * 16 + lane_c
    lane_f, above_f = _find_bin(hist[pl.ds(bc * 16, 16)], above_c, k)
    return bc * 16 + lane_f, above_f


def _sc_thresholds(xi):
    mesh = plsc.VectorSubcoreMesh(core_axis_name="c", subcore_axis_name="s")

    @functools.partial(
        pl.kernel, mesh=mesh,
        compiler_params=pltpu.CompilerParams(needs_layout_passes=False),
        out_type=jax.ShapeDtypeStruct((_B, 16), jnp.int32),
        scratch_types=[
            pltpu.VMEM((2, _CHUNK, _C), jnp.int32),
            pltpu.VMEM((65536,), jnp.float32),
            pltpu.VMEM((4096,), jnp.float32),
            pltpu.VMEM((16,), jnp.int32),
            pltpu.SemaphoreType.DMA,
            pltpu.SemaphoreType.DMA,
        ],
    )
    def sck(xi_hbm, o_hbm, buf, hist, hist_c, ov, sem0, sem1):
        wid = lax.axis_index("s") * 2 + lax.axis_index("c")
        sems = (sem0, sem1)
        zf = jnp.zeros((16,), jnp.float32)

        def zero_hists():
            def zb(i, _):
                for u in range(8):
                    hist[pl.ds(i * 128 + u * 16, 16)] = zf
                return ()
            lax.fori_loop(0, 512, zb, ())

            def zc(i, _):
                for u in range(8):
                    hist_c[pl.ds(i * 128 + u * 16, 16)] = zf
                return ()
            lax.fori_loop(0, 32, zc, ())

        def row_pass(r, fine_sel, k):
            zero_hists()
            ones = jnp.ones((16,), jnp.float32)

            def start(c):
                return pltpu.async_copy(
                    xi_hbm.at[r, pl.ds(c * _CHUNK, _CHUNK), :],
                    buf.at[c % 2], sems[c % 2])

            h = {0: start(0)}
            for c in range(_NCH):
                h[c].wait()
                if c + 1 < _NCH:
                    h[c + 1] = start(c + 1)
                cur = c % 2

                def ibody(i, _):
                    for j in range(_C // 16):
                        v = buf[cur, i, pl.ds(j * 16, 16)]
                        key = jnp.where(v < 0, v ^ jnp.int32(0x7FFFFFFF), v)
                        b1 = (key >> 16) + jnp.int32(32768)
                        if fine_sel is None:
                            plsc.addupdate_scatter(hist, (b1,), ones)
                            plsc.addupdate_scatter(hist_c, (b1 >> 4,), ones)
                        else:
                            bins = key & jnp.int32(0xFFFF)
                            m = b1 == fine_sel
                            plsc.addupdate_scatter(hist, (bins,), ones,
                                                   mask=m)
                            plsc.addupdate_scatter(hist_c, (bins >> 4,),
                                                   ones, mask=m)
                    return ()
                lax.fori_loop(0, _CHUNK, ibody, ())
            return _scan_hists(hist, hist_c, k)

        def process_row(i, _):
            r = wid + _NW * i

            @pl.when(r < _B)
            def _():
                b1, above_a = row_pass(r, None, jnp.float32(_K))
                lo, _above2 = row_pass(r, b1, jnp.float32(_K) - above_a)
                kth = ((b1 - 32768) << 16) | lo
                ov[...] = jnp.broadcast_to(kth, (16,))
                pltpu.sync_copy(ov, o_hbm.at[r])
            return ()

        lax.fori_loop(0, 3, process_row, ())

    return sck(xi)


def _lif_body(thr_ref, x_ref, o_ref):
    j = pl.program_id(0)
    u = jnp.zeros((_HW_CHUNK, _C), jnp.float32)
    for t in range(_TIMESTEP):
        xt = x_ref[t, 0]
        kth_f = thr_ref[t * _BS + j]
        mask = (xt >= kth_f).astype(jnp.float32)
        spk_prev = (u > _VTH).astype(jnp.float32)
        u = _TAU * u * (1.0 - spk_prev) + xt
        s = (u > _VTH).astype(jnp.float32)
        o_ref[t, 0] = s * mask


def kernel(x):
    xp = x.transpose(0, 2, 3, 1).reshape(_B, _HW, _C)
    xi = jax.lax.bitcast_convert_type(xp, jnp.int32)
    kth = _sc_thresholds(xi)[:, 0]
    kth_f = jax.lax.bitcast_convert_type(
        jnp.where(kth < 0, kth ^ jnp.int32(0x7FFFFFFF), kth), jnp.float32)

    x4 = xp.reshape(_TIMESTEP, _BS, _HW, _C)
    out = pl.pallas_call(
        _lif_body,
        grid_spec=pltpu.PrefetchScalarGridSpec(
            num_scalar_prefetch=1,
            grid=(_BS, _NCHUNK),
            in_specs=[pl.BlockSpec((_TIMESTEP, 1, _HW_CHUNK, _C),
                                   lambda j, c, *_: (0, j, c, 0))],
            out_specs=pl.BlockSpec((_TIMESTEP, 1, _HW_CHUNK, _C),
                                   lambda j, c, *_: (0, j, c, 0)),
        ),
        out_shape=jax.ShapeDtypeStruct((_TIMESTEP, _BS, _HW, _C),
                                       jnp.float32),
    )(kth_f, x4)
    return out.reshape(_B, _H, _W, _C).transpose(0, 3, 1, 2)
